# baseline (speedup 1.0000x reference)
import functools

import jax
import jax.numpy as jnp
from jax import lax
from jax.experimental import pallas as pl
from jax.experimental.pallas import tpu as pltpu
from jax.experimental.pallas import tpu_sc as plsc

N = 10000
E = 320000
IN_DIM = 128
HID = 64
ZD = 32

NC = 2
NS = 16
NW = NC * NS
CH = 128
NCHUNK = 80
EPAD = NW * NCHUNK * CH
NPAD = 10240
RPS = NPAD // NS
DEGW = 8
NB = 4


def _worker_id():
    return lax.axis_index("s") * NC + lax.axis_index("c")


def _make_deg_kernel():
    mesh = plsc.VectorSubcoreMesh(core_axis_name="c", subcore_axis_name="s")

    @functools.partial(
        pl.kernel,
        mesh=mesh,
        out_type=jax.ShapeDtypeStruct((NC, NPAD, DEGW), jnp.float32),
        scratch_types=[
            pltpu.VMEM((NCHUNK, CH), jnp.int32),
            pltpu.VMEM((CH, DEGW), jnp.float32),
            pltpu.VMEM_SHARED((NPAD, DEGW), jnp.float32),
            pltpu.SemaphoreType.DMA,
            pltpu.SemaphoreType.DMA,
        ],
        compiler_params=pltpu.CompilerParams(use_tc_tiling_on_sc=False),
    )
    def deg_kernel(dst3_hbm, ones_hbm, zero_hbm, out_hbm,
                   didx_all, ones_v, deg_sh, sem_a, sem_b):
        c = lax.axis_index("c")
        s = lax.axis_index("s")
        w = _worker_id()
        pltpu.sync_copy(dst3_hbm.at[w], didx_all)
        pltpu.sync_copy(ones_hbm, ones_v)
        pltpu.sync_copy(zero_hbm, deg_sh.at[pl.ds(s * RPS, RPS)])
        plsc.subcore_barrier()

        pltpu.async_copy(ones_v, deg_sh.at[didx_all.at[0]], sem_a, add=True)
        pltpu.async_copy(ones_v, deg_sh.at[didx_all.at[1]], sem_b, add=True)

        def body(j, carry):
            pltpu.make_async_copy(ones_v, deg_sh.at[didx_all.at[0]],
                                  sem_a).wait()
            pltpu.async_copy(ones_v, deg_sh.at[didx_all.at[2 * j]],
                             sem_a, add=True)
            pltpu.make_async_copy(ones_v, deg_sh.at[didx_all.at[0]],
                                  sem_b).wait()
            pltpu.async_copy(ones_v, deg_sh.at[didx_all.at[2 * j + 1]],
                             sem_b, add=True)
            return carry

        lax.fori_loop(1, NCHUNK // 2, body, 0)
        pltpu.make_async_copy(ones_v, deg_sh.at[didx_all.at[0]], sem_a).wait()
        pltpu.make_async_copy(ones_v, deg_sh.at[didx_all.at[0]], sem_b).wait()
        plsc.subcore_barrier()
        pltpu.sync_copy(deg_sh.at[pl.ds(s * RPS, RPS)],
                        out_hbm.at[c, pl.ds(s * RPS, RPS)])

    return deg_kernel


def _make_mp_kernel(D):
    mesh = plsc.VectorSubcoreMesh(core_axis_name="c", subcore_axis_name="s")

    @functools.partial(
        pl.kernel,
        mesh=mesh,
        out_type=jax.ShapeDtypeStruct((NC, NPAD, D), jnp.float32),
        scratch_types=[
            pltpu.VMEM((NCHUNK, CH), jnp.int32),
            pltpu.VMEM((NCHUNK, CH), jnp.int32),
            [pltpu.VMEM((CH, D), jnp.float32)] * NB,
            pltpu.VMEM_SHARED((NPAD, D), jnp.float32),
            [pltpu.SemaphoreType.DMA] * NB,
            [pltpu.SemaphoreType.DMA] * NB,
        ],
        compiler_params=pltpu.CompilerParams(use_tc_tiling_on_sc=False),
    )
    def mp_kernel(g_hbm, src3_hbm, dst3_hbm, zero_hbm, out_hbm,
                  sidx_all, didx_all, rows, acc_sh, sem_g, sem_s):
        c = lax.axis_index("c")
        s = lax.axis_index("s")
        w = _worker_id()
        pltpu.sync_copy(src3_hbm.at[w], sidx_all)
        pltpu.sync_copy(dst3_hbm.at[w], didx_all)
        pltpu.sync_copy(zero_hbm, acc_sh.at[pl.ds(s * RPS, RPS)])
        plsc.subcore_barrier()

        def wait_g(b):
            pltpu.make_async_copy(g_hbm.at[sidx_all.at[0]], rows[b],
                                  sem_g[b]).wait()

        def wait_s(b):
            pltpu.make_async_copy(rows[b], acc_sh.at[didx_all.at[0]],
                                  sem_s[b]).wait()

        pltpu.async_copy(g_hbm.at[sidx_all.at[0]], rows[0], sem_g[0])

        def body(j, carry):
            i0 = 2 * j
            pltpu.async_copy(g_hbm.at[sidx_all.at[i0 + 1]], rows[1],
                             sem_g[1])
            wait_g(0)
            pltpu.sync_copy(rows[0], acc_sh.at[didx_all.at[i0]], add=True)
            pltpu.async_copy(g_hbm.at[sidx_all.at[i0 + 2]], rows[0],
                             sem_g[0])
            wait_g(1)
            pltpu.sync_copy(rows[1], acc_sh.at[didx_all.at[i0 + 1]],
                            add=True)
            return carry

        lax.fori_loop(0, NCHUNK // 2 - 1, body, 0)
        pltpu.async_copy(g_hbm.at[sidx_all.at[NCHUNK - 1]], rows[1],
                         sem_g[1])
        wait_g(0)
        pltpu.sync_copy(rows[0], acc_sh.at[didx_all.at[NCHUNK - 2]],
                        add=True)
        wait_g(1)
        pltpu.sync_copy(rows[1], acc_sh.at[didx_all.at[NCHUNK - 1]],
                        add=True)
        plsc.subcore_barrier()
        pltpu.sync_copy(acc_sh.at[pl.ds(s * RPS, RPS)],
                        out_hbm.at[c, pl.ds(s * RPS, RPS)])

    return mp_kernel


@functools.lru_cache(maxsize=None)
def _sc_kernels():
    return _make_deg_kernel(), _make_mp_kernel(HID), _make_mp_kernel(ZD)


R = 1000


def _tc1_body(x_ref, w1_ref, degp_ref, g_ref, dinv_ref):
    cnt = degp_ref[0, :, 0:1] + degp_ref[1, :, 0:1]
    dinv = lax.rsqrt(cnt + 1.0)
    h = jnp.dot(x_ref[...], w1_ref[...],
                preferred_element_type=jnp.float32)
    g_ref[...] = h * dinv
    dinv_ref[...] = dinv


def _tc2_body(accp_ref, g1_ref, dinv_ref, b1_ref, w2_ref, g2_ref):
    dinv = dinv_ref[...]
    a = (accp_ref[0] + accp_ref[1] + g1_ref[...]) * dinv + b1_ref[...]
    h = jnp.maximum(a, 0.0)
    g2_ref[...] = jnp.dot(h, w2_ref[...],
                          preferred_element_type=jnp.float32) * dinv


def _tc3_body(accp_ref, g2_ref, dinv_ref, b2_ref,
              wmu_ref, bmu_ref, wlv_ref, blv_ref, mu_ref, lv_ref):
    dinv = dinv_ref[...]
    hf = (accp_ref[0] + accp_ref[1] + g2_ref[...]) * dinv + b2_ref[...]
    mu_ref[...] = jnp.dot(hf, wmu_ref[...],
                          preferred_element_type=jnp.float32) + bmu_ref[...]
    lv_ref[...] = jnp.dot(hf, wlv_ref[...],
                          preferred_element_type=jnp.float32) + blv_ref[...]


def _row_spec(d):
    return pl.BlockSpec((R, d), lambda i: (i, 0))


def _full_spec(shape):
    nd = len(shape)
    return pl.BlockSpec(shape, lambda i: (0,) * nd)


def _tc1(x, W1, degp):
    return pl.pallas_call(
        _tc1_body,
        grid=(N // R,),
        in_specs=[
            _row_spec(IN_DIM),
            _full_spec((IN_DIM, HID)),
            pl.BlockSpec((NC, R, DEGW), lambda i: (0, i, 0)),
        ],
        out_specs=[_row_spec(HID), _row_spec(1)],
        out_shape=[
            jax.ShapeDtypeStruct((NPAD, HID), jnp.float32),
            jax.ShapeDtypeStruct((N, 1), jnp.float32),
        ],
    )(x, W1, degp)


def _tc2(accp, g1, dinv, b1, W2):
    return pl.pallas_call(
        _tc2_body,
        grid=(N // R,),
        in_specs=[
            pl.BlockSpec((NC, R, HID), lambda i: (0, i, 0)),
            _row_spec(HID),
            _row_spec(1),
            _full_spec((1, HID)),
            _full_spec((HID, ZD)),
        ],
        out_specs=_row_spec(ZD),
        out_shape=jax.ShapeDtypeStruct((NPAD, ZD), jnp.float32),
    )(accp, g1, dinv, b1, W2)


def _tc3(accp, g2, dinv, b2, Wmu, bmu, Wlv, blv):
    return pl.pallas_call(
        _tc3_body,
        grid=(N // R,),
        in_specs=[
            pl.BlockSpec((NC, R, ZD), lambda i: (0, i, 0)),
            _row_spec(ZD),
            _row_spec(1),
            _full_spec((1, ZD)),
            _full_spec((ZD, ZD)),
            _full_spec((1, ZD)),
            _full_spec((ZD, ZD)),
            _full_spec((1, ZD)),
        ],
        out_specs=[_row_spec(ZD), _row_spec(ZD)],
        out_shape=[
            jax.ShapeDtypeStruct((N, ZD), jnp.float32),
            jax.ShapeDtypeStruct((N, ZD), jnp.float32),
        ],
    )(accp, g2, dinv, b2, Wmu, bmu, Wlv, blv)


def kernel(x, edge_index, W1, b1, W2, b2, Wmu, bmu, Wlv, blv):
    pad = jnp.full((EPAD - E,), NPAD - 1, jnp.int32)
    src3 = jnp.concatenate([edge_index[0], pad]).reshape(NW, NCHUNK, CH)
    dst3 = jnp.concatenate([edge_index[1], pad]).reshape(NW, NCHUNK, CH)
    ones_deg = jnp.ones((CH, DEGW), jnp.float32)
    zero_deg = jnp.zeros((RPS, DEGW), jnp.float32)
    zero_hid = jnp.zeros((RPS, HID), jnp.float32)
    zero_z = jnp.zeros((RPS, ZD), jnp.float32)

    _deg_sc, _mp_sc_hid, _mp_sc_z = _sc_kernels()
    degp = _deg_sc(dst3, ones_deg, zero_deg)
    g1, dinv = _tc1(x, W1, degp)
    accp1 = _mp_sc_hid(g1, src3, dst3, zero_hid)
    g2 = _tc2(accp1, g1, dinv, b1.reshape(1, HID), W2)
    accp2 = _mp_sc_z(g2, src3, dst3, zero_z)
    mu, lv = _tc3(accp2, g2, dinv, b2.reshape(1, ZD),
                  Wmu, bmu.reshape(1, ZD), Wlv, blv.reshape(1, ZD))
    return (mu, lv)


# speedup vs baseline: 1.9752x; 1.9752x over previous
import functools

import jax
import jax.numpy as jnp
from jax import lax
from jax.experimental import pallas as pl
from jax.experimental.pallas import tpu as pltpu
from jax.experimental.pallas import tpu_sc as plsc

N = 10000
E = 320000
IN_DIM = 128
HID = 64
ZD = 32

NC = 2
NS = 16
NW = NC * NS
CH = 128
NCHUNK = 80
EPAD = NW * NCHUNK * CH
NPAD = 10240
RPS = NPAD // NS
DEGW = 8
NB = 4


def _worker_id():
    return lax.axis_index("s") * NC + lax.axis_index("c")


def _make_deg_kernel():
    mesh = plsc.VectorSubcoreMesh(core_axis_name="c", subcore_axis_name="s")

    @functools.partial(
        pl.kernel,
        mesh=mesh,
        out_type=jax.ShapeDtypeStruct((NC, NPAD, DEGW), jnp.float32),
        scratch_types=[
            pltpu.VMEM((NCHUNK, CH), jnp.int32),
            pltpu.VMEM((CH, DEGW), jnp.float32),
            pltpu.VMEM_SHARED((NPAD, DEGW), jnp.float32),
            pltpu.SemaphoreType.DMA,
            pltpu.SemaphoreType.DMA,
        ],
        compiler_params=pltpu.CompilerParams(use_tc_tiling_on_sc=False),
    )
    def deg_kernel(dst3_hbm, ones_hbm, zero_hbm, out_hbm,
                   didx_all, ones_v, deg_sh, sem_a, sem_b):
        c = lax.axis_index("c")
        s = lax.axis_index("s")
        w = _worker_id()
        pltpu.sync_copy(dst3_hbm.at[w], didx_all)
        pltpu.sync_copy(ones_hbm, ones_v)
        pltpu.sync_copy(zero_hbm, deg_sh.at[pl.ds(s * RPS, RPS)])
        plsc.subcore_barrier()

        pltpu.async_copy(ones_v, deg_sh.at[didx_all.at[0]], sem_a, add=True)
        pltpu.async_copy(ones_v, deg_sh.at[didx_all.at[1]], sem_b, add=True)

        def body(j, carry):
            pltpu.make_async_copy(ones_v, deg_sh.at[didx_all.at[0]],
                                  sem_a).wait()
            pltpu.async_copy(ones_v, deg_sh.at[didx_all.at[2 * j]],
                             sem_a, add=True)
            pltpu.make_async_copy(ones_v, deg_sh.at[didx_all.at[0]],
                                  sem_b).wait()
            pltpu.async_copy(ones_v, deg_sh.at[didx_all.at[2 * j + 1]],
                             sem_b, add=True)
            return carry

        lax.fori_loop(1, NCHUNK // 2, body, 0)
        pltpu.make_async_copy(ones_v, deg_sh.at[didx_all.at[0]], sem_a).wait()
        pltpu.make_async_copy(ones_v, deg_sh.at[didx_all.at[0]], sem_b).wait()
        plsc.subcore_barrier()
        pltpu.sync_copy(deg_sh.at[pl.ds(s * RPS, RPS)],
                        out_hbm.at[c, pl.ds(s * RPS, RPS)])

    return deg_kernel


def _make_mp_kernel(D):
    mesh = plsc.VectorSubcoreMesh(core_axis_name="c", subcore_axis_name="s")

    @functools.partial(
        pl.kernel,
        mesh=mesh,
        out_type=jax.ShapeDtypeStruct((NC, NPAD, D), jnp.float32),
        scratch_types=[
            pltpu.VMEM((NCHUNK, CH), jnp.int32),
            pltpu.VMEM((NCHUNK, CH), jnp.int32),
            [pltpu.VMEM((CH, D), jnp.float32)] * NB,
            pltpu.VMEM_SHARED((NPAD, D), jnp.float32),
            [pltpu.SemaphoreType.DMA] * NB,
            [pltpu.SemaphoreType.DMA] * NB,
        ],
        compiler_params=pltpu.CompilerParams(use_tc_tiling_on_sc=False),
    )
    def mp_kernel(g_hbm, src3_hbm, dst3_hbm, zero_hbm, out_hbm,
                  sidx_all, didx_all, rows, acc_sh, sem_g, sem_s):
        c = lax.axis_index("c")
        s = lax.axis_index("s")
        w = _worker_id()
        pltpu.sync_copy(src3_hbm.at[w], sidx_all)
        pltpu.sync_copy(dst3_hbm.at[w], didx_all)
        pltpu.sync_copy(zero_hbm, acc_sh.at[pl.ds(s * RPS, RPS)])
        plsc.subcore_barrier()

        def wait_g(b):
            pltpu.make_async_copy(g_hbm.at[sidx_all.at[0]], rows[b],
                                  sem_g[b]).wait()

        def wait_s(b):
            pltpu.make_async_copy(rows[b], acc_sh.at[didx_all.at[0]],
                                  sem_s[b]).wait()

        pltpu.async_copy(g_hbm.at[sidx_all.at[0]], rows[0], sem_g[0])

        def body(j, carry):
            i0 = 2 * j
            pltpu.async_copy(g_hbm.at[sidx_all.at[i0 + 1]], rows[1],
                             sem_g[1])
            wait_g(0)
            pltpu.sync_copy(rows[0], acc_sh.at[didx_all.at[i0]], add=True)
            pltpu.async_copy(g_hbm.at[sidx_all.at[i0 + 2]], rows[0],
                             sem_g[0])
            wait_g(1)
            pltpu.sync_copy(rows[1], acc_sh.at[didx_all.at[i0 + 1]],
                            add=True)
            return carry

        lax.fori_loop(0, NCHUNK // 2 - 1, body, 0)
        pltpu.async_copy(g_hbm.at[sidx_all.at[NCHUNK - 1]], rows[1],
                         sem_g[1])
        wait_g(0)
        pltpu.sync_copy(rows[0], acc_sh.at[didx_all.at[NCHUNK - 2]],
                        add=True)
        wait_g(1)
        pltpu.sync_copy(rows[1], acc_sh.at[didx_all.at[NCHUNK - 1]],
                        add=True)
        plsc.subcore_barrier()
        pltpu.sync_copy(acc_sh.at[pl.ds(s * RPS, RPS)],
                        out_hbm.at[c, pl.ds(s * RPS, RPS)])

    return mp_kernel


@functools.lru_cache(maxsize=None)
def _sc_kernels():
    return _make_deg_kernel(), _make_mp_kernel(HID), _make_mp_kernel(ZD)


R = 1000


def _tc1_body(x_ref, w1_ref, degp_ref, g_ref, dinv_ref):
    cnt = degp_ref[0, :, 0:1] + degp_ref[1, :, 0:1]
    dinv = lax.rsqrt(cnt + 1.0)
    h = jnp.dot(x_ref[...], w1_ref[...],
                preferred_element_type=jnp.float32)
    g_ref[...] = h * dinv
    dinv_ref[...] = dinv


def _tc2_body(accp_ref, g1_ref, dinv_ref, b1_ref, w2_ref, g2_ref):
    dinv = dinv_ref[...]
    a = (accp_ref[0] + accp_ref[1] + g1_ref[...]) * dinv + b1_ref[...]
    h = jnp.maximum(a, 0.0)
    g2_ref[...] = jnp.dot(h, w2_ref[...],
                          preferred_element_type=jnp.float32) * dinv


def _tc3_body(accp_ref, g2_ref, dinv_ref, b2_ref,
              wmu_ref, bmu_ref, wlv_ref, blv_ref, mu_ref, lv_ref):
    dinv = dinv_ref[...]
    hf = (accp_ref[0] + accp_ref[1] + g2_ref[...]) * dinv + b2_ref[...]
    mu_ref[...] = jnp.dot(hf, wmu_ref[...],
                          preferred_element_type=jnp.float32) + bmu_ref[...]
    lv_ref[...] = jnp.dot(hf, wlv_ref[...],
                          preferred_element_type=jnp.float32) + blv_ref[...]


def _row_spec(d):
    return pl.BlockSpec((R, d), lambda i: (i, 0))


def _full_spec(shape):
    nd = len(shape)
    return pl.BlockSpec(shape, lambda i: (0,) * nd)


def _tc1(x, W1, degp):
    return pl.pallas_call(
        _tc1_body,
        grid=(N // R,),
        in_specs=[
            _row_spec(IN_DIM),
            _full_spec((IN_DIM, HID)),
            pl.BlockSpec((NC, R, DEGW), lambda i: (0, i, 0)),
        ],
        out_specs=[_row_spec(HID), _row_spec(1)],
        out_shape=[
            jax.ShapeDtypeStruct((NPAD, HID), jnp.float32),
            jax.ShapeDtypeStruct((N, 1), jnp.float32),
        ],
    )(x, W1, degp)


def _tc2(accp, g1, dinv, b1, W2):
    return pl.pallas_call(
        _tc2_body,
        grid=(N // R,),
        in_specs=[
            pl.BlockSpec((NC, R, HID), lambda i: (0, i, 0)),
            _row_spec(HID),
            _row_spec(1),
            _full_spec((1, HID)),
            _full_spec((HID, ZD)),
        ],
        out_specs=_row_spec(ZD),
        out_shape=jax.ShapeDtypeStruct((NPAD, ZD), jnp.float32),
    )(accp, g1, dinv, b1, W2)


def _tc3(accp, g2, dinv, b2, Wmu, bmu, Wlv, blv):
    return pl.pallas_call(
        _tc3_body,
        grid=(N // R,),
        in_specs=[
            pl.BlockSpec((NC, R, ZD), lambda i: (0, i, 0)),
            _row_spec(ZD),
            _row_spec(1),
            _full_spec((1, ZD)),
            _full_spec((ZD, ZD)),
            _full_spec((1, ZD)),
            _full_spec((ZD, ZD)),
            _full_spec((1, ZD)),
        ],
        out_specs=[_row_spec(ZD), _row_spec(ZD)],
        out_shape=[
            jax.ShapeDtypeStruct((N, ZD), jnp.float32),
            jax.ShapeDtypeStruct((N, ZD), jnp.float32),
        ],
    )(accp, g2, dinv, b2, Wmu, bmu, Wlv, blv)


def kernel(x, edge_index, W1, b1, W2, b2, Wmu, bmu, Wlv, blv):
    pad = N + (jnp.arange(EPAD - E, dtype=jnp.int32) % (NPAD - N))
    src3 = jnp.concatenate([edge_index[0], pad]).reshape(NW, NCHUNK, CH)
    dst3 = jnp.concatenate([edge_index[1], pad]).reshape(NW, NCHUNK, CH)
    ones_deg = jnp.ones((CH, DEGW), jnp.float32)
    zero_deg = jnp.zeros((RPS, DEGW), jnp.float32)
    zero_hid = jnp.zeros((RPS, HID), jnp.float32)
    zero_z = jnp.zeros((RPS, ZD), jnp.float32)

    _deg_sc, _mp_sc_hid, _mp_sc_z = _sc_kernels()
    degp = _deg_sc(dst3, ones_deg, zero_deg)
    g1, dinv = _tc1(x, W1, degp)
    accp1 = _mp_sc_hid(g1, src3, dst3, zero_hid)
    g2 = _tc2(accp1, g1, dinv, b1.reshape(1, HID), W2)
    accp2 = _mp_sc_z(g2, src3, dst3, zero_z)
    mu, lv = _tc3(accp2, g2, dinv, b2.reshape(1, ZD),
                  Wmu, bmu.reshape(1, ZD), Wlv, blv.reshape(1, ZD))
    return (mu, lv)
